# baseline (speedup 1.0000x reference)
import functools
import jax
import jax.numpy as jnp
from jax import lax
from jax.experimental import pallas as pl
from jax.experimental.pallas import tpu as pltpu, tpu_sc as plsc

N = 10000
E = 160000
D_IN = 5189
DIM = 256
HALF = 128
NC = 2
NS = 16
CH = 128

EC_T = E // NS
EC_CHUNKS = 79
EC_PAD = EC_CHUNKS * CH

EH_T = E // (NC * NS)
EH_CHUNKS = 40
EH_PAD = EH_CHUNKS * CH

SI_CHUNKS = 3
SI_PAD = SI_CHUNKS * CH

SR_CHUNKS = 5
NROW_PAD = NS * SR_CHUNKS * CH

NACC = 10240
RPT = NACC // NS

_mesh = plsc.VectorSubcoreMesh(core_axis_name="c", subcore_axis_name="s")

f32 = jnp.float32


def _dot_hi(a, b):
    return jnp.dot(a, b, preferred_element_type=f32,
                   precision=jax.lax.Precision.HIGHEST)


@functools.partial(
    pl.kernel,
    out_type=jax.ShapeDtypeStruct((NC, 5, NACC, 16), f32),
    mesh=_mesh,
    scratch_types=dict(
        acc=[pltpu.VMEM_SHARED((NACC, 16), f32) for _ in range(5)],
        onesv=pltpu.VMEM((CH, 16), f32),
        edgev=pltpu.VMEM((EH_CHUNKS, CH), jnp.int32),
        segv=pltpu.VMEM((SI_CHUNKS, CH), jnp.int32),
        zv=pltpu.VMEM((CH, 16), f32),
    ),
)
def _hist_kernel(dsts, segs, out, acc, onesv, edgev, segv, zv):
    c = lax.axis_index("c")
    s = lax.axis_index("s")

    @pl.loop(0, CH)
    def _(i):
        onesv[i, :] = jnp.ones((16,), f32)
        zv[i, :] = jnp.zeros((16,), f32)

    for a in range(5):
        for z in range(RPT // CH):
            pltpu.sync_copy(zv, acc[a].at[pl.ds(s * RPT + z * CH, CH)])
    plsc.subcore_barrier()

    for j in range(3):
        pltpu.sync_copy(dsts.at[j, c, s], edgev)

        @pl.loop(0, EH_CHUNKS)
        def _(k):
            pltpu.sync_copy(onesv, acc[j].at[edgev.at[k]], add=True)

    for m in range(2):
        pltpu.sync_copy(segs.at[m, c, s], segv)

        @pl.loop(0, SI_CHUNKS)
        def _(k):
            pltpu.sync_copy(onesv, acc[3 + m].at[segv.at[k]], add=True)

    plsc.subcore_barrier()
    for a in range(5):
        pltpu.sync_copy(acc[a].at[pl.ds(s * RPT, RPT)],
                        out.at[c, a, pl.ds(s * RPT, RPT)])


@functools.partial(
    pl.kernel,
    out_type=jax.ShapeDtypeStruct((6, NACC, HALF), f32),
    mesh=_mesh,
    scratch_types=dict(
        acc=pltpu.VMEM_SHARED((NACC, HALF), f32),
        srcv=pltpu.VMEM((EC_CHUNKS, CH), jnp.int32),
        dstv=pltpu.VMEM((EC_CHUNKS, CH), jnp.int32),
        rows=pltpu.VMEM((CH, HALF), f32),
        gsem=pltpu.SemaphoreType.DMA,
    ),
)
def _conv_kernel(g_all, src_pad, dst_pad, out, acc, srcv, dstv, rows, gsem):
    c = lax.axis_index("c")
    s = lax.axis_index("s")

    for j in range(3):
        @pl.loop(0, CH)
        def _(i):
            @pl.loop(0, HALF // 16)
            def _(q):
                rows[i, pl.ds(q * 16, 16)] = jnp.zeros((16,), f32)

        for z in range(RPT // CH):
            pltpu.sync_copy(rows, acc.at[pl.ds(s * RPT + z * CH, CH)])
        pltpu.sync_copy(src_pad.at[j, s], srcv)
        pltpu.sync_copy(dst_pad.at[j, s], dstv)
        plsc.subcore_barrier()

        @pl.loop(0, EC_CHUNKS)
        def _(k):
            pltpu.async_copy(g_all.at[2 * j + c].at[srcv.at[k]], rows,
                             gsem).wait()
            pltpu.sync_copy(rows, acc.at[dstv.at[k]], add=True)

        plsc.subcore_barrier()
        pltpu.sync_copy(acc.at[pl.ds(s * RPT, RPT)],
                        out.at[2 * j + c, pl.ds(s * RPT, RPT)])
        plsc.subcore_barrier()


@functools.partial(
    pl.kernel,
    out_type=jax.ShapeDtypeStruct((2, NC, NACC, HALF), f32),
    mesh=_mesh,
    scratch_types=dict(
        acc=pltpu.VMEM_SHARED((NACC, HALF), f32),
        idxv=pltpu.VMEM((NS * SR_CHUNKS, CH), jnp.int32),
        rows=pltpu.VMEM((CH, HALF), f32),
    ),
)
def _segmean_kernel(h3pad, segidx, out, acc, idxv, rows):
    c = lax.axis_index("c")
    s = lax.axis_index("s")

    for k in range(2):
        @pl.loop(0, CH)
        def _(i):
            @pl.loop(0, HALF // 16)
            def _(q):
                rows[i, pl.ds(q * 16, 16)] = jnp.zeros((16,), f32)

        for z in range(RPT // CH):
            pltpu.sync_copy(rows, acc.at[pl.ds(s * RPT + z * CH, CH)])
        pltpu.sync_copy(segidx.at[k], idxv)
        plsc.subcore_barrier()

        @pl.loop(0, SR_CHUNKS)
        def _(m):
            pltpu.sync_copy(
                h3pad.at[c, pl.ds(s * SR_CHUNKS * CH + m * CH, CH)], rows)
            pltpu.sync_copy(rows, acc.at[idxv.at[s * SR_CHUNKS + m]], add=True)

        plsc.subcore_barrier()
        pltpu.sync_copy(acc.at[pl.ds(s * RPT, RPT)],
                        out.at[k, c, pl.ds(s * RPT, RPT)])
        plsc.subcore_barrier()


RB = 400
GRID = N // RB


def _dinvs(hist_blk):
    cnt = hist_blk[0] + hist_blk[1]
    return [1.0 / jnp.sqrt(cnt[j, :, 0] + 1.0) for j in range(3)]


def _matmul1_body(x_ref, w_ref, hist_ref, g_ref):
    h = jnp.dot(x_ref[...], w_ref[...], preferred_element_type=f32)
    dinv = _dinvs(hist_ref[...])
    for j in range(3):
        gj = h[:, j * DIM:(j + 1) * DIM] * dinv[j][:, None]
        g_ref[2 * j, :, :] = gj[:, :HALF]
        g_ref[2 * j + 1, :, :] = gj[:, HALF:]


def _stage1_matmul(x, w_all, hist):
    return pl.pallas_call(
        _matmul1_body,
        grid=(GRID,),
        in_specs=[
            pl.BlockSpec((RB, D_IN), lambda i: (i, 0)),
            pl.BlockSpec((D_IN, 3 * DIM), lambda i: (0, 0)),
            pl.BlockSpec((NC, 5, RB, 16), lambda i: (0, 0, i, 0)),
        ],
        out_specs=pl.BlockSpec((6, RB, HALF), lambda i: (0, i, 0)),
        out_shape=jax.ShapeDtypeStruct((6, N, HALF), f32),
    )(x, w_all, hist)


def _epilogue_body(bias_ref, acc_ref, g_ref, hist_ref, wa_ref, ba_ref,
                   wb_ref, bb_ref, wn_ref, g2_ref):
    dinv = _dinvs(hist_ref[...])
    acc = acc_ref[...]
    g = g_ref[...]
    xs = []
    for j in range(3):
        full = jnp.concatenate([acc[2 * j] + g[2 * j],
                                acc[2 * j + 1] + g[2 * j + 1]], axis=1)
        xs.append(jax.nn.relu(full * dinv[j][:, None] + bias_ref[j, :][None, :]))
    xcat = jnp.concatenate(xs, axis=1)
    a = jax.nn.relu(_dot_hi(xcat, wa_ref[...])
                    + ba_ref[...][None, :])
    h2 = _dot_hi(a, wb_ref[...]) + bb_ref[...][None, :]
    h2b = _dot_hi(h2, wn_ref[...])
    dinv2 = _dinvs(hist_ref[...])
    for j in range(3):
        g2 = h2b[:, j * DIM:(j + 1) * DIM] * dinv2[j][:, None]
        g2_ref[2 * j, :, :] = g2[:, :HALF]
        g2_ref[2 * j + 1, :, :] = g2[:, HALF:]


def _stage_epilogue(bias3, acc_all, g_all, hist, wa, ba, wb, bb, w_next):
    return pl.pallas_call(
        _epilogue_body,
        grid=(GRID,),
        in_specs=[
            pl.BlockSpec((3, DIM), lambda i: (0, 0)),
            pl.BlockSpec((6, RB, HALF), lambda i: (0, i, 0)),
            pl.BlockSpec((6, RB, HALF), lambda i: (0, i, 0)),
            pl.BlockSpec((NC, 5, RB, 16), lambda i: (0, 0, i, 0)),
            pl.BlockSpec((3 * DIM, DIM), lambda i: (0, 0)),
            pl.BlockSpec((DIM,), lambda i: (0,)),
            pl.BlockSpec((DIM, DIM), lambda i: (0, 0)),
            pl.BlockSpec((DIM,), lambda i: (0,)),
            pl.BlockSpec((DIM, 3 * DIM), lambda i: (0, 0)),
        ],
        out_specs=pl.BlockSpec((6, RB, HALF), lambda i: (0, i, 0)),
        out_shape=jax.ShapeDtypeStruct((6, N, HALF), f32),
    )(bias3, acc_all, g_all, hist, wa, ba, wb, bb, w_next)


def _epilogue2_body(bias_ref, acc_ref, g_ref, hist_ref, wa_ref, ba_ref,
                    wb_ref, bb_ref, h3_ref):
    dinv = _dinvs(hist_ref[...])
    acc = acc_ref[...]
    g = g_ref[...]
    xs = []
    for j in range(3):
        full = jnp.concatenate([acc[2 * j] + g[2 * j],
                                acc[2 * j + 1] + g[2 * j + 1]], axis=1)
        xs.append(jax.nn.relu(full * dinv[j][:, None] + bias_ref[j, :][None, :]))
    xcat = jnp.concatenate(xs, axis=1)
    a = jax.nn.relu(_dot_hi(xcat, wa_ref[...])
                    + ba_ref[...][None, :])
    h3 = _dot_hi(a, wb_ref[...]) + bb_ref[...][None, :]
    h3_ref[0, :, :] = h3[:, :HALF]
    h3_ref[1, :, :] = h3[:, HALF:]


def _stage_epilogue2(bias3, acc_all, g_all, hist, wa, ba, wb, bb):
    return pl.pallas_call(
        _epilogue2_body,
        grid=(GRID,),
        in_specs=[
            pl.BlockSpec((3, DIM), lambda i: (0, 0)),
            pl.BlockSpec((6, RB, HALF), lambda i: (0, i, 0)),
            pl.BlockSpec((6, RB, HALF), lambda i: (0, i, 0)),
            pl.BlockSpec((NC, 5, RB, 16), lambda i: (0, 0, i, 0)),
            pl.BlockSpec((3 * DIM, DIM), lambda i: (0, 0)),
            pl.BlockSpec((DIM,), lambda i: (0,)),
            pl.BlockSpec((DIM, DIM), lambda i: (0, 0)),
            pl.BlockSpec((DIM,), lambda i: (0,)),
        ],
        out_specs=pl.BlockSpec((NC, RB, HALF), lambda i: (0, i, 0)),
        out_shape=jax.ShapeDtypeStruct((NC, NROW_PAD, HALF), f32),
    )(bias3, acc_all, g_all, hist, wa, ba, wb, bb)


def _mlp3_body(sacc_ref, hist_ref, wa_ref, ba_ref, wb_ref, bb_ref, o_ref):
    hist_blk = hist_ref[...]
    cnt = hist_blk[0] + hist_blk[1]
    c1 = jnp.maximum(cnt[3, :, 0], 1.0)
    c3 = jnp.maximum(cnt[4, :, 0], 1.0)
    sacc = sacc_ref[...]
    s1 = jnp.concatenate([sacc[0, 0], sacc[0, 1]], axis=1) / c1[:, None]
    s3 = jnp.concatenate([sacc[1, 0], sacc[1, 1]], axis=1) / c3[:, None]
    xcat = jnp.concatenate([s1, s3, s3], axis=1)
    a = jax.nn.relu(_dot_hi(xcat, wa_ref[...])
                    + ba_ref[...][None, :])
    o = _dot_hi(a, wb_ref[...]) + bb_ref[...][None, :]
    m = jnp.max(o, axis=1, keepdims=True)
    lse = m + jnp.log(jnp.sum(jnp.exp(o - m), axis=1, keepdims=True))
    o_ref[...] = o - lse


def _stage_mlp3(sacc, hist, wa, ba, wb, bb):
    return pl.pallas_call(
        _mlp3_body,
        grid=(GRID,),
        in_specs=[
            pl.BlockSpec((2, NC, RB, HALF), lambda i: (0, 0, i, 0)),
            pl.BlockSpec((NC, 5, RB, 16), lambda i: (0, 0, i, 0)),
            pl.BlockSpec((3 * DIM, DIM), lambda i: (0, 0)),
            pl.BlockSpec((DIM,), lambda i: (0,)),
            pl.BlockSpec((DIM, 7), lambda i: (0, 0)),
            pl.BlockSpec((7,), lambda i: (0,)),
        ],
        out_specs=pl.BlockSpec((RB, 7), lambda i: (i, 0)),
        out_shape=jax.ShapeDtypeStruct((N, 7), f32),
    )(sacc, hist, wa, ba, wb, bb)


def _pad_conv_edges(edge_index):
    src, dst = edge_index[0], edge_index[1]
    pad = EC_PAD - EC_T
    src2 = src.reshape(NS, EC_T)
    dst2 = dst.reshape(NS, EC_T)
    fill_src = (jnp.arange(pad, dtype=jnp.int32) * 89) % N
    fill_dst = N + (jnp.arange(pad, dtype=jnp.int32) % NS)
    src_pad = jnp.concatenate(
        [src2, jnp.broadcast_to(fill_src, (NS, pad))], axis=1)
    dst_pad = jnp.concatenate(
        [dst2, jnp.broadcast_to(fill_dst, (NS, pad))], axis=1)
    return (src_pad.reshape(NS, EC_CHUNKS, CH),
            dst_pad.reshape(NS, EC_CHUNKS, CH))


def _pad_hist(dst):
    pad = EH_PAD - EH_T
    d = dst.reshape(NC * NS, EH_T)
    fill = N + (jnp.arange(pad, dtype=jnp.int32) % NS)
    d = jnp.concatenate([d, jnp.broadcast_to(fill, (NC * NS, pad))], axis=1)
    return d.reshape(NC, NS, EH_CHUNKS, CH)


def _pad_seg_hist(idx):
    total = NC * NS * SI_PAD
    fill = N + (jnp.arange(total - N, dtype=jnp.int32) % NS)
    d = jnp.concatenate([idx, fill])
    return d.reshape(NC, NS, SI_CHUNKS, CH)


def _pad_seg_scatter(idx):
    fill = N + (jnp.arange(NROW_PAD - N, dtype=jnp.int32) % NS)
    d = jnp.concatenate([idx, fill])
    return d.reshape(NS * SR_CHUNKS, CH)


def kernel(x, edge_index_1, edge_index_2, edge_index_3, index_1, index_2,
           index_3, W1_1, b1_1, W1_2, b1_2, W1_3, b1_3, mlp1_Wa, mlp1_ba,
           mlp1_Wb, mlp1_bb, W2_1, b2_1, W2_2, b2_2, W2_3, b2_3, mlp2_Wa,
           mlp2_ba, mlp2_Wb, mlp2_bb, mlp3_Wa, mlp3_ba, mlp3_Wb, mlp3_bb):
    w1_all = jnp.concatenate([W1_1, W1_2, W1_3], axis=1)
    w2_all = jnp.concatenate([W2_1, W2_2, W2_3], axis=1)
    b1_all = jnp.stack([b1_1, b1_2, b1_3])
    b2_all = jnp.stack([b2_1, b2_2, b2_3])

    edges = [edge_index_1, edge_index_2, edge_index_3]
    srcs, dsts = zip(*[_pad_conv_edges(e) for e in edges])
    src_pad = jnp.stack(srcs)
    dst_pad = jnp.stack(dsts)
    hist_dsts = jnp.stack([_pad_hist(e[1]) for e in edges])
    hist_segs = jnp.stack([_pad_seg_hist(index_1),
                           _pad_seg_hist(index_3)])
    seg_scatter = jnp.stack([_pad_seg_scatter(index_1),
                             _pad_seg_scatter(index_3)])

    hist = _hist_kernel(hist_dsts, hist_segs)
    g1 = _stage1_matmul(x, w1_all, hist)
    acc1 = _conv_kernel(g1, src_pad, dst_pad)
    g2 = _stage_epilogue(b1_all, acc1, g1, hist, mlp1_Wa, mlp1_ba,
                         mlp1_Wb, mlp1_bb, w2_all)
    acc2 = _conv_kernel(g2, src_pad, dst_pad)
    h3pad = _stage_epilogue2(b2_all, acc2, g2, hist, mlp2_Wa, mlp2_ba,
                             mlp2_Wb, mlp2_bb)
    sacc = _segmean_kernel(h3pad, seg_scatter)
    return _stage_mlp3(sacc, hist, mlp3_Wa, mlp3_ba, mlp3_Wb, mlp3_bb)


# speedup vs baseline: 1.0915x; 1.0915x over previous
import functools
import jax
import jax.numpy as jnp
from jax import lax
from jax.experimental import pallas as pl
from jax.experimental.pallas import tpu as pltpu, tpu_sc as plsc

N = 10000
E = 160000
D_IN = 5189
DIM = 256
HALF = 128
NC = 2
NS = 16
CH = 128

EC_T = E // NS
EC_CHUNKS = 80
EC_PAD = EC_CHUNKS * CH
EC_HALF = EC_CHUNKS // 2

EH_T = E // (NC * NS)
EH_CHUNKS = 40
EH_PAD = EH_CHUNKS * CH

SI_CHUNKS = 3
SI_PAD = SI_CHUNKS * CH

SR_CHUNKS = 5
NROW_PAD = NS * SR_CHUNKS * CH

NACC = 10240
RPT = NACC // NS

_mesh = plsc.VectorSubcoreMesh(core_axis_name="c", subcore_axis_name="s")

f32 = jnp.float32


def _dot_hi(a, b):
    return jnp.dot(a, b, preferred_element_type=f32,
                   precision=jax.lax.Precision.HIGHEST)


@functools.partial(
    pl.kernel,
    out_type=jax.ShapeDtypeStruct((NC, 5, NACC, 16), f32),
    mesh=_mesh,
    scratch_types=dict(
        acc=[pltpu.VMEM_SHARED((NACC, 16), f32) for _ in range(5)],
        onesv=pltpu.VMEM((CH, 16), f32),
        edgev=pltpu.VMEM((EH_CHUNKS, CH), jnp.int32),
        segv=pltpu.VMEM((SI_CHUNKS, CH), jnp.int32),
        zv=pltpu.VMEM((CH, 16), f32),
    ),
)
def _hist_kernel(dsts, segs, out, acc, onesv, edgev, segv, zv):
    c = lax.axis_index("c")
    s = lax.axis_index("s")

    @pl.loop(0, CH)
    def _(i):
        onesv[i, :] = jnp.ones((16,), f32)
        zv[i, :] = jnp.zeros((16,), f32)

    for a in range(5):
        for z in range(RPT // CH):
            pltpu.sync_copy(zv, acc[a].at[pl.ds(s * RPT + z * CH, CH)])
    plsc.subcore_barrier()

    for j in range(3):
        pltpu.sync_copy(dsts.at[j, c, s], edgev)

        @pl.loop(0, EH_CHUNKS)
        def _(k):
            pltpu.sync_copy(onesv, acc[j].at[edgev.at[k]], add=True)

    for m in range(2):
        pltpu.sync_copy(segs.at[m, c, s], segv)

        @pl.loop(0, SI_CHUNKS)
        def _(k):
            pltpu.sync_copy(onesv, acc[3 + m].at[segv.at[k]], add=True)

    plsc.subcore_barrier()
    for a in range(5):
        pltpu.sync_copy(acc[a].at[pl.ds(s * RPT, RPT)],
                        out.at[c, a, pl.ds(s * RPT, RPT)])


@functools.partial(
    pl.kernel,
    out_type=jax.ShapeDtypeStruct((6, NACC, HALF), f32),
    mesh=_mesh,
    scratch_types=dict(
        acc=pltpu.VMEM_SHARED((NACC, HALF), f32),
        srcv=pltpu.VMEM((EC_HALF, CH), jnp.int32),
        dstv=pltpu.VMEM((EC_HALF, CH), jnp.int32),
        rows0=pltpu.VMEM((CH, HALF), f32),
        rows1=pltpu.VMEM((CH, HALF), f32),
        sem0=pltpu.SemaphoreType.DMA,
        sem1=pltpu.SemaphoreType.DMA,
    ),
)
def _conv_kernel(g_all, src_pad, dst_pad, out, acc, srcv, dstv, rows0, rows1,
                 sem0, sem1):
    c = lax.axis_index("c")
    s = lax.axis_index("s")
    bufs = (rows0, rows1)
    sems = (sem0, sem1)

    for j in range(3):
        @pl.loop(0, CH)
        def _(i):
            @pl.loop(0, HALF // 16)
            def _(q):
                rows0[i, pl.ds(q * 16, 16)] = jnp.zeros((16,), f32)

        for z in range(RPT // CH):
            pltpu.sync_copy(rows0, acc.at[pl.ds(s * RPT + z * CH, CH)])
        plsc.subcore_barrier()

        g_hbm = g_all.at[2 * j + c]
        for h in range(2):
            pltpu.sync_copy(src_pad.at[j, s, pl.ds(h * EC_HALF, EC_HALF)],
                            srcv)
            pltpu.sync_copy(dst_pad.at[j, s, pl.ds(h * EC_HALF, EC_HALF)],
                            dstv)
            @pl.loop(0, EC_HALF // 2)
            def _(g):
                descs = [
                    pltpu.async_copy(g_hbm.at[srcv.at[2 * g + b]], bufs[b],
                                     sems[b])
                    for b in range(2)
                ]
                for b in range(2):
                    descs[b].wait()
                for b in range(2):
                    pltpu.sync_copy(bufs[b], acc.at[dstv.at[2 * g + b]],
                                    add=True)

        plsc.subcore_barrier()
        pltpu.sync_copy(acc.at[pl.ds(s * RPT, RPT)],
                        out.at[2 * j + c, pl.ds(s * RPT, RPT)])
        plsc.subcore_barrier()


@functools.partial(
    pl.kernel,
    out_type=jax.ShapeDtypeStruct((2, NC, NACC, HALF), f32),
    mesh=_mesh,
    scratch_types=dict(
        acc=pltpu.VMEM_SHARED((NACC, HALF), f32),
        idxv=pltpu.VMEM((NS * SR_CHUNKS, CH), jnp.int32),
        rows=pltpu.VMEM((CH, HALF), f32),
    ),
)
def _segmean_kernel(h3pad, segidx, out, acc, idxv, rows):
    c = lax.axis_index("c")
    s = lax.axis_index("s")

    for k in range(2):
        @pl.loop(0, CH)
        def _(i):
            @pl.loop(0, HALF // 16)
            def _(q):
                rows[i, pl.ds(q * 16, 16)] = jnp.zeros((16,), f32)

        for z in range(RPT // CH):
            pltpu.sync_copy(rows, acc.at[pl.ds(s * RPT + z * CH, CH)])
        pltpu.sync_copy(segidx.at[k], idxv)
        plsc.subcore_barrier()

        @pl.loop(0, SR_CHUNKS)
        def _(m):
            pltpu.sync_copy(
                h3pad.at[c, pl.ds(s * SR_CHUNKS * CH + m * CH, CH)], rows)
            pltpu.sync_copy(rows, acc.at[idxv.at[s * SR_CHUNKS + m]], add=True)

        plsc.subcore_barrier()
        pltpu.sync_copy(acc.at[pl.ds(s * RPT, RPT)],
                        out.at[k, c, pl.ds(s * RPT, RPT)])
        plsc.subcore_barrier()


RB = 400
GRID = N // RB


def _dinvs(hist_blk):
    cnt = hist_blk[0] + hist_blk[1]
    return [1.0 / jnp.sqrt(cnt[j, :, 0] + 1.0) for j in range(3)]


def _matmul1_body(x_ref, w_ref, hist_ref, g_ref):
    h = jnp.dot(x_ref[...], w_ref[...], preferred_element_type=f32)
    dinv = _dinvs(hist_ref[...])
    for j in range(3):
        gj = h[:, j * DIM:(j + 1) * DIM] * dinv[j][:, None]
        g_ref[2 * j, :, :] = gj[:, :HALF]
        g_ref[2 * j + 1, :, :] = gj[:, HALF:]


def _stage1_matmul(x, w_all, hist):
    return pl.pallas_call(
        _matmul1_body,
        grid=(GRID,),
        in_specs=[
            pl.BlockSpec((RB, D_IN), lambda i: (i, 0)),
            pl.BlockSpec((D_IN, 3 * DIM), lambda i: (0, 0)),
            pl.BlockSpec((NC, 5, RB, 16), lambda i: (0, 0, i, 0)),
        ],
        out_specs=pl.BlockSpec((6, RB, HALF), lambda i: (0, i, 0)),
        out_shape=jax.ShapeDtypeStruct((6, N, HALF), f32),
    )(x, w_all, hist)


def _epilogue_body(bias_ref, acc_ref, g_ref, hist_ref, wa_ref, ba_ref,
                   wb_ref, bb_ref, wn_ref, g2_ref):
    dinv = _dinvs(hist_ref[...])
    acc = acc_ref[...]
    g = g_ref[...]
    xs = []
    for j in range(3):
        full = jnp.concatenate([acc[2 * j] + g[2 * j],
                                acc[2 * j + 1] + g[2 * j + 1]], axis=1)
        xs.append(jax.nn.relu(full * dinv[j][:, None] + bias_ref[j, :][None, :]))
    xcat = jnp.concatenate(xs, axis=1)
    a = jax.nn.relu(_dot_hi(xcat, wa_ref[...])
                    + ba_ref[...][None, :])
    h2 = _dot_hi(a, wb_ref[...]) + bb_ref[...][None, :]
    h2b = _dot_hi(h2, wn_ref[...])
    dinv2 = _dinvs(hist_ref[...])
    for j in range(3):
        g2 = h2b[:, j * DIM:(j + 1) * DIM] * dinv2[j][:, None]
        g2_ref[2 * j, :, :] = g2[:, :HALF]
        g2_ref[2 * j + 1, :, :] = g2[:, HALF:]


def _stage_epilogue(bias3, acc_all, g_all, hist, wa, ba, wb, bb, w_next):
    return pl.pallas_call(
        _epilogue_body,
        grid=(GRID,),
        in_specs=[
            pl.BlockSpec((3, DIM), lambda i: (0, 0)),
            pl.BlockSpec((6, RB, HALF), lambda i: (0, i, 0)),
            pl.BlockSpec((6, RB, HALF), lambda i: (0, i, 0)),
            pl.BlockSpec((NC, 5, RB, 16), lambda i: (0, 0, i, 0)),
            pl.BlockSpec((3 * DIM, DIM), lambda i: (0, 0)),
            pl.BlockSpec((DIM,), lambda i: (0,)),
            pl.BlockSpec((DIM, DIM), lambda i: (0, 0)),
            pl.BlockSpec((DIM,), lambda i: (0,)),
            pl.BlockSpec((DIM, 3 * DIM), lambda i: (0, 0)),
        ],
        out_specs=pl.BlockSpec((6, RB, HALF), lambda i: (0, i, 0)),
        out_shape=jax.ShapeDtypeStruct((6, N, HALF), f32),
    )(bias3, acc_all, g_all, hist, wa, ba, wb, bb, w_next)


def _epilogue2_body(bias_ref, acc_ref, g_ref, hist_ref, wa_ref, ba_ref,
                    wb_ref, bb_ref, h3_ref):
    dinv = _dinvs(hist_ref[...])
    acc = acc_ref[...]
    g = g_ref[...]
    xs = []
    for j in range(3):
        full = jnp.concatenate([acc[2 * j] + g[2 * j],
                                acc[2 * j + 1] + g[2 * j + 1]], axis=1)
        xs.append(jax.nn.relu(full * dinv[j][:, None] + bias_ref[j, :][None, :]))
    xcat = jnp.concatenate(xs, axis=1)
    a = jax.nn.relu(_dot_hi(xcat, wa_ref[...])
                    + ba_ref[...][None, :])
    h3 = _dot_hi(a, wb_ref[...]) + bb_ref[...][None, :]
    h3_ref[0, :, :] = h3[:, :HALF]
    h3_ref[1, :, :] = h3[:, HALF:]


def _stage_epilogue2(bias3, acc_all, g_all, hist, wa, ba, wb, bb):
    return pl.pallas_call(
        _epilogue2_body,
        grid=(GRID,),
        in_specs=[
            pl.BlockSpec((3, DIM), lambda i: (0, 0)),
            pl.BlockSpec((6, RB, HALF), lambda i: (0, i, 0)),
            pl.BlockSpec((6, RB, HALF), lambda i: (0, i, 0)),
            pl.BlockSpec((NC, 5, RB, 16), lambda i: (0, 0, i, 0)),
            pl.BlockSpec((3 * DIM, DIM), lambda i: (0, 0)),
            pl.BlockSpec((DIM,), lambda i: (0,)),
            pl.BlockSpec((DIM, DIM), lambda i: (0, 0)),
            pl.BlockSpec((DIM,), lambda i: (0,)),
        ],
        out_specs=pl.BlockSpec((NC, RB, HALF), lambda i: (0, i, 0)),
        out_shape=jax.ShapeDtypeStruct((NC, NROW_PAD, HALF), f32),
    )(bias3, acc_all, g_all, hist, wa, ba, wb, bb)


def _mlp3_body(sacc_ref, hist_ref, wa_ref, ba_ref, wb_ref, bb_ref, o_ref):
    hist_blk = hist_ref[...]
    cnt = hist_blk[0] + hist_blk[1]
    c1 = jnp.maximum(cnt[3, :, 0], 1.0)
    c3 = jnp.maximum(cnt[4, :, 0], 1.0)
    sacc = sacc_ref[...]
    s1 = jnp.concatenate([sacc[0, 0], sacc[0, 1]], axis=1) / c1[:, None]
    s3 = jnp.concatenate([sacc[1, 0], sacc[1, 1]], axis=1) / c3[:, None]
    xcat = jnp.concatenate([s1, s3, s3], axis=1)
    a = jax.nn.relu(_dot_hi(xcat, wa_ref[...])
                    + ba_ref[...][None, :])
    o = _dot_hi(a, wb_ref[...]) + bb_ref[...][None, :]
    m = jnp.max(o, axis=1, keepdims=True)
    lse = m + jnp.log(jnp.sum(jnp.exp(o - m), axis=1, keepdims=True))
    o_ref[...] = o - lse


def _stage_mlp3(sacc, hist, wa, ba, wb, bb):
    return pl.pallas_call(
        _mlp3_body,
        grid=(GRID,),
        in_specs=[
            pl.BlockSpec((2, NC, RB, HALF), lambda i: (0, 0, i, 0)),
            pl.BlockSpec((NC, 5, RB, 16), lambda i: (0, 0, i, 0)),
            pl.BlockSpec((3 * DIM, DIM), lambda i: (0, 0)),
            pl.BlockSpec((DIM,), lambda i: (0,)),
            pl.BlockSpec((DIM, 7), lambda i: (0, 0)),
            pl.BlockSpec((7,), lambda i: (0,)),
        ],
        out_specs=pl.BlockSpec((RB, 7), lambda i: (i, 0)),
        out_shape=jax.ShapeDtypeStruct((N, 7), f32),
    )(sacc, hist, wa, ba, wb, bb)


def _pad_conv_edges(edge_index):
    src, dst = edge_index[0], edge_index[1]
    pad = EC_PAD - EC_T
    src2 = src.reshape(NS, EC_T)
    dst2 = dst.reshape(NS, EC_T)
    fill_src = (jnp.arange(pad, dtype=jnp.int32) * 89) % N
    fill_dst = N + (jnp.arange(pad, dtype=jnp.int32) % NS)
    src_pad = jnp.concatenate(
        [src2, jnp.broadcast_to(fill_src, (NS, pad))], axis=1)
    dst_pad = jnp.concatenate(
        [dst2, jnp.broadcast_to(fill_dst, (NS, pad))], axis=1)
    return (src_pad.reshape(NS, EC_CHUNKS, CH),
            dst_pad.reshape(NS, EC_CHUNKS, CH))


def _pad_hist(dst):
    pad = EH_PAD - EH_T
    d = dst.reshape(NC * NS, EH_T)
    fill = N + (jnp.arange(pad, dtype=jnp.int32) % NS)
    d = jnp.concatenate([d, jnp.broadcast_to(fill, (NC * NS, pad))], axis=1)
    return d.reshape(NC, NS, EH_CHUNKS, CH)


def _pad_seg_hist(idx):
    total = NC * NS * SI_PAD
    fill = N + (jnp.arange(total - N, dtype=jnp.int32) % NS)
    d = jnp.concatenate([idx, fill])
    return d.reshape(NC, NS, SI_CHUNKS, CH)


def _pad_seg_scatter(idx):
    fill = N + (jnp.arange(NROW_PAD - N, dtype=jnp.int32) % NS)
    d = jnp.concatenate([idx, fill])
    return d.reshape(NS * SR_CHUNKS, CH)


def kernel(x, edge_index_1, edge_index_2, edge_index_3, index_1, index_2,
           index_3, W1_1, b1_1, W1_2, b1_2, W1_3, b1_3, mlp1_Wa, mlp1_ba,
           mlp1_Wb, mlp1_bb, W2_1, b2_1, W2_2, b2_2, W2_3, b2_3, mlp2_Wa,
           mlp2_ba, mlp2_Wb, mlp2_bb, mlp3_Wa, mlp3_ba, mlp3_Wb, mlp3_bb):
    w1_all = jnp.concatenate([W1_1, W1_2, W1_3], axis=1)
    w2_all = jnp.concatenate([W2_1, W2_2, W2_3], axis=1)
    b1_all = jnp.stack([b1_1, b1_2, b1_3])
    b2_all = jnp.stack([b2_1, b2_2, b2_3])

    edges = [edge_index_1, edge_index_2, edge_index_3]
    srcs, dsts = zip(*[_pad_conv_edges(e) for e in edges])
    src_pad = jnp.stack(srcs)
    dst_pad = jnp.stack(dsts)
    hist_dsts = jnp.stack([_pad_hist(e[1]) for e in edges])
    hist_segs = jnp.stack([_pad_seg_hist(index_1),
                           _pad_seg_hist(index_3)])
    seg_scatter = jnp.stack([_pad_seg_scatter(index_1),
                             _pad_seg_scatter(index_3)])

    hist = _hist_kernel(hist_dsts, hist_segs)
    g1 = _stage1_matmul(x, w1_all, hist)
    acc1 = _conv_kernel(g1, src_pad, dst_pad)
    g2 = _stage_epilogue(b1_all, acc1, g1, hist, mlp1_Wa, mlp1_ba,
                         mlp1_Wb, mlp1_bb, w2_all)
    acc2 = _conv_kernel(g2, src_pad, dst_pad)
    h3pad = _stage_epilogue2(b2_all, acc2, g2, hist, mlp2_Wa, mlp2_ba,
                             mlp2_Wb, mlp2_bb)
    sacc = _segmean_kernel(h3pad, seg_scatter)
    return _stage_mlp3(sacc, hist, mlp3_Wa, mlp3_ba, mlp3_Wb, mlp3_bb)


# speedup vs baseline: 1.0949x; 1.0031x over previous
import functools
import jax
import jax.numpy as jnp
from jax import lax
from jax.experimental import pallas as pl
from jax.experimental.pallas import tpu as pltpu, tpu_sc as plsc

N = 10000
E = 160000
D_IN = 5189
DIM = 256
HALF = 128
NC = 2
NS = 16
CH = 128

EC_T = E // NS
EC_CHUNKS = 80
EC_PAD = EC_CHUNKS * CH
EC_HALF = EC_CHUNKS // 2

EH_T = E // (NC * NS)
EH_CHUNKS = 40
EH_PAD = EH_CHUNKS * CH

SI_CHUNKS = 3
SI_PAD = SI_CHUNKS * CH

SR_CHUNKS = 5
NROW_PAD = NS * SR_CHUNKS * CH

NACC = 10240
RPT = NACC // NS

_mesh = plsc.VectorSubcoreMesh(core_axis_name="c", subcore_axis_name="s")

f32 = jnp.float32


def _dot_hi(a, b):
    return jnp.dot(a, b, preferred_element_type=f32,
                   precision=jax.lax.Precision.HIGHEST)


@functools.partial(
    pl.kernel,
    out_type=jax.ShapeDtypeStruct((NC, 5, NACC, 16), f32),
    mesh=_mesh,
    scratch_types=dict(
        acc=[pltpu.VMEM_SHARED((NACC, 16), f32) for _ in range(5)],
        onesv=pltpu.VMEM((CH, 16), f32),
        edgev=pltpu.VMEM((EH_CHUNKS, CH), jnp.int32),
        segv=pltpu.VMEM((SI_CHUNKS, CH), jnp.int32),
        zv=pltpu.VMEM((CH, 16), f32),
    ),
)
def _hist_kernel(dsts, segs, out, acc, onesv, edgev, segv, zv):
    c = lax.axis_index("c")
    s = lax.axis_index("s")

    @pl.loop(0, CH)
    def _(i):
        onesv[i, :] = jnp.ones((16,), f32)
        zv[i, :] = jnp.zeros((16,), f32)

    for a in range(5):
        for z in range(RPT // CH):
            pltpu.sync_copy(zv, acc[a].at[pl.ds(s * RPT + z * CH, CH)])
    plsc.subcore_barrier()

    for j in range(3):
        pltpu.sync_copy(dsts.at[j, c, s], edgev)

        @pl.loop(0, EH_CHUNKS)
        def _(k):
            pltpu.sync_copy(onesv, acc[j].at[edgev.at[k]], add=True)

    for m in range(2):
        pltpu.sync_copy(segs.at[m, c, s], segv)

        @pl.loop(0, SI_CHUNKS)
        def _(k):
            pltpu.sync_copy(onesv, acc[3 + m].at[segv.at[k]], add=True)

    plsc.subcore_barrier()
    for a in range(5):
        pltpu.sync_copy(acc[a].at[pl.ds(s * RPT, RPT)],
                        out.at[c, a, pl.ds(s * RPT, RPT)])


@functools.partial(
    pl.kernel,
    out_type=jax.ShapeDtypeStruct((6, NACC, HALF), f32),
    mesh=_mesh,
    scratch_types=dict(
        acc=pltpu.VMEM_SHARED((NACC, HALF), f32),
        srcv=pltpu.VMEM((EC_HALF, CH), jnp.int32),
        dstv=pltpu.VMEM((EC_HALF, CH), jnp.int32),
        rows0=pltpu.VMEM((CH, HALF), f32),
        rows1=pltpu.VMEM((CH, HALF), f32),
        sem0=pltpu.SemaphoreType.DMA,
        sem1=pltpu.SemaphoreType.DMA,
    ),
)
def _conv_kernel(g_all, src_pad, dst_pad, out, acc, srcv, dstv, rows0, rows1,
                 sem0, sem1):
    c = lax.axis_index("c")
    s = lax.axis_index("s")
    bufs = (rows0, rows1)
    sems = (sem0, sem1)

    for j in range(3):
        @pl.loop(0, CH)
        def _(i):
            @pl.loop(0, HALF // 16)
            def _(q):
                rows0[i, pl.ds(q * 16, 16)] = jnp.zeros((16,), f32)

        for z in range(RPT // CH):
            pltpu.sync_copy(rows0, acc.at[pl.ds(s * RPT + z * CH, CH)])
        plsc.subcore_barrier()

        g_hbm = g_all.at[2 * j + c]
        for h in range(2):
            pltpu.sync_copy(src_pad.at[j, s, pl.ds(h * EC_HALF, EC_HALF)],
                            srcv)
            pltpu.sync_copy(dst_pad.at[j, s, pl.ds(h * EC_HALF, EC_HALF)],
                            dstv)
            @pl.loop(0, EC_HALF // 2)
            def _(g):
                descs = [
                    pltpu.async_copy(g_hbm.at[srcv.at[2 * g + b]], bufs[b],
                                     sems[b])
                    for b in range(2)
                ]
                for b in range(2):
                    descs[b].wait()
                for b in range(2):
                    pltpu.sync_copy(bufs[b], acc.at[dstv.at[2 * g + b]],
                                    add=True)

        plsc.subcore_barrier()
        pltpu.sync_copy(acc.at[pl.ds(s * RPT, RPT)],
                        out.at[2 * j + c, pl.ds(s * RPT, RPT)])
        plsc.subcore_barrier()


@functools.partial(
    pl.kernel,
    out_type=jax.ShapeDtypeStruct((2, NC, NACC, HALF), f32),
    mesh=_mesh,
    scratch_types=dict(
        acc=pltpu.VMEM_SHARED((NACC, HALF), f32),
        idxv=pltpu.VMEM((NS * SR_CHUNKS, CH), jnp.int32),
        rows=pltpu.VMEM((CH, HALF), f32),
    ),
)
def _segmean_kernel(h3pad, segidx, out, acc, idxv, rows):
    c = lax.axis_index("c")
    s = lax.axis_index("s")

    for k in range(2):
        @pl.loop(0, CH)
        def _(i):
            @pl.loop(0, HALF // 16)
            def _(q):
                rows[i, pl.ds(q * 16, 16)] = jnp.zeros((16,), f32)

        for z in range(RPT // CH):
            pltpu.sync_copy(rows, acc.at[pl.ds(s * RPT + z * CH, CH)])
        pltpu.sync_copy(segidx.at[k], idxv)
        plsc.subcore_barrier()

        @pl.loop(0, SR_CHUNKS)
        def _(m):
            pltpu.sync_copy(
                h3pad.at[c, pl.ds(s * SR_CHUNKS * CH + m * CH, CH)], rows)
            pltpu.sync_copy(rows, acc.at[idxv.at[s * SR_CHUNKS + m]], add=True)

        plsc.subcore_barrier()
        pltpu.sync_copy(acc.at[pl.ds(s * RPT, RPT)],
                        out.at[k, c, pl.ds(s * RPT, RPT)])
        plsc.subcore_barrier()


RB = 400
GRID = N // RB


def _dinvs(hist_blk):
    cnt = hist_blk[0] + hist_blk[1]
    return [1.0 / jnp.sqrt(cnt[j, :, 0] + 1.0) for j in range(3)]


def _matmul1_body(x_ref, w_ref, h_ref):
    h = jnp.dot(x_ref[...], w_ref[...], preferred_element_type=f32)
    for j in range(3):
        h_ref[2 * j, :, :] = h[:, j * DIM:j * DIM + HALF]
        h_ref[2 * j + 1, :, :] = h[:, j * DIM + HALF:(j + 1) * DIM]


def _stage1_matmul(x, w_all):
    return pl.pallas_call(
        _matmul1_body,
        grid=(GRID,),
        in_specs=[
            pl.BlockSpec((RB, D_IN), lambda i: (i, 0)),
            pl.BlockSpec((D_IN, 3 * DIM), lambda i: (0, 0)),
        ],
        out_specs=pl.BlockSpec((6, RB, HALF), lambda i: (0, i, 0)),
        out_shape=jax.ShapeDtypeStruct((6, N, HALF), f32),
    )(x, w_all)


def _scale_body(h_ref, hist_ref, g_ref):
    dinv = _dinvs(hist_ref[...])
    h = h_ref[...]
    for j in range(3):
        for cc in range(2):
            g_ref[2 * j + cc, :, :] = h[2 * j + cc] * dinv[j][:, None]


def _stage1_scale(h_all, hist):
    return pl.pallas_call(
        _scale_body,
        grid=(GRID,),
        in_specs=[
            pl.BlockSpec((6, RB, HALF), lambda i: (0, i, 0)),
            pl.BlockSpec((NC, 5, RB, 16), lambda i: (0, 0, i, 0)),
        ],
        out_specs=pl.BlockSpec((6, RB, HALF), lambda i: (0, i, 0)),
        out_shape=jax.ShapeDtypeStruct((6, N, HALF), f32),
    )(h_all, hist)


def _epilogue_body(bias_ref, acc_ref, g_ref, hist_ref, wa_ref, ba_ref,
                   wb_ref, bb_ref, wn_ref, g2_ref):
    dinv = _dinvs(hist_ref[...])
    acc = acc_ref[...]
    g = g_ref[...]
    xs = []
    for j in range(3):
        full = jnp.concatenate([acc[2 * j] + g[2 * j],
                                acc[2 * j + 1] + g[2 * j + 1]], axis=1)
        xs.append(jax.nn.relu(full * dinv[j][:, None] + bias_ref[j, :][None, :]))
    xcat = jnp.concatenate(xs, axis=1)
    a = jax.nn.relu(_dot_hi(xcat, wa_ref[...])
                    + ba_ref[...][None, :])
    h2 = _dot_hi(a, wb_ref[...]) + bb_ref[...][None, :]
    h2b = _dot_hi(h2, wn_ref[...])
    dinv2 = _dinvs(hist_ref[...])
    for j in range(3):
        g2 = h2b[:, j * DIM:(j + 1) * DIM] * dinv2[j][:, None]
        g2_ref[2 * j, :, :] = g2[:, :HALF]
        g2_ref[2 * j + 1, :, :] = g2[:, HALF:]


def _stage_epilogue(bias3, acc_all, g_all, hist, wa, ba, wb, bb, w_next):
    return pl.pallas_call(
        _epilogue_body,
        grid=(GRID,),
        in_specs=[
            pl.BlockSpec((3, DIM), lambda i: (0, 0)),
            pl.BlockSpec((6, RB, HALF), lambda i: (0, i, 0)),
            pl.BlockSpec((6, RB, HALF), lambda i: (0, i, 0)),
            pl.BlockSpec((NC, 5, RB, 16), lambda i: (0, 0, i, 0)),
            pl.BlockSpec((3 * DIM, DIM), lambda i: (0, 0)),
            pl.BlockSpec((DIM,), lambda i: (0,)),
            pl.BlockSpec((DIM, DIM), lambda i: (0, 0)),
            pl.BlockSpec((DIM,), lambda i: (0,)),
            pl.BlockSpec((DIM, 3 * DIM), lambda i: (0, 0)),
        ],
        out_specs=pl.BlockSpec((6, RB, HALF), lambda i: (0, i, 0)),
        out_shape=jax.ShapeDtypeStruct((6, N, HALF), f32),
    )(bias3, acc_all, g_all, hist, wa, ba, wb, bb, w_next)


def _epilogue2_body(bias_ref, acc_ref, g_ref, hist_ref, wa_ref, ba_ref,
                    wb_ref, bb_ref, h3_ref):
    dinv = _dinvs(hist_ref[...])
    acc = acc_ref[...]
    g = g_ref[...]
    xs = []
    for j in range(3):
        full = jnp.concatenate([acc[2 * j] + g[2 * j],
                                acc[2 * j + 1] + g[2 * j + 1]], axis=1)
        xs.append(jax.nn.relu(full * dinv[j][:, None] + bias_ref[j, :][None, :]))
    xcat = jnp.concatenate(xs, axis=1)
    a = jax.nn.relu(_dot_hi(xcat, wa_ref[...])
                    + ba_ref[...][None, :])
    h3 = _dot_hi(a, wb_ref[...]) + bb_ref[...][None, :]
    h3_ref[0, :, :] = h3[:, :HALF]
    h3_ref[1, :, :] = h3[:, HALF:]


def _stage_epilogue2(bias3, acc_all, g_all, hist, wa, ba, wb, bb):
    return pl.pallas_call(
        _epilogue2_body,
        grid=(GRID,),
        in_specs=[
            pl.BlockSpec((3, DIM), lambda i: (0, 0)),
            pl.BlockSpec((6, RB, HALF), lambda i: (0, i, 0)),
            pl.BlockSpec((6, RB, HALF), lambda i: (0, i, 0)),
            pl.BlockSpec((NC, 5, RB, 16), lambda i: (0, 0, i, 0)),
            pl.BlockSpec((3 * DIM, DIM), lambda i: (0, 0)),
            pl.BlockSpec((DIM,), lambda i: (0,)),
            pl.BlockSpec((DIM, DIM), lambda i: (0, 0)),
            pl.BlockSpec((DIM,), lambda i: (0,)),
        ],
        out_specs=pl.BlockSpec((NC, RB, HALF), lambda i: (0, i, 0)),
        out_shape=jax.ShapeDtypeStruct((NC, NROW_PAD, HALF), f32),
    )(bias3, acc_all, g_all, hist, wa, ba, wb, bb)


def _mlp3_body(sacc_ref, hist_ref, wa_ref, ba_ref, wb_ref, bb_ref, o_ref):
    hist_blk = hist_ref[...]
    cnt = hist_blk[0] + hist_blk[1]
    c1 = jnp.maximum(cnt[3, :, 0], 1.0)
    c3 = jnp.maximum(cnt[4, :, 0], 1.0)
    sacc = sacc_ref[...]
    s1 = jnp.concatenate([sacc[0, 0], sacc[0, 1]], axis=1) / c1[:, None]
    s3 = jnp.concatenate([sacc[1, 0], sacc[1, 1]], axis=1) / c3[:, None]
    xcat = jnp.concatenate([s1, s3, s3], axis=1)
    a = jax.nn.relu(_dot_hi(xcat, wa_ref[...])
                    + ba_ref[...][None, :])
    o = _dot_hi(a, wb_ref[...]) + bb_ref[...][None, :]
    m = jnp.max(o, axis=1, keepdims=True)
    lse = m + jnp.log(jnp.sum(jnp.exp(o - m), axis=1, keepdims=True))
    o_ref[...] = o - lse


def _stage_mlp3(sacc, hist, wa, ba, wb, bb):
    return pl.pallas_call(
        _mlp3_body,
        grid=(GRID,),
        in_specs=[
            pl.BlockSpec((2, NC, RB, HALF), lambda i: (0, 0, i, 0)),
            pl.BlockSpec((NC, 5, RB, 16), lambda i: (0, 0, i, 0)),
            pl.BlockSpec((3 * DIM, DIM), lambda i: (0, 0)),
            pl.BlockSpec((DIM,), lambda i: (0,)),
            pl.BlockSpec((DIM, 7), lambda i: (0, 0)),
            pl.BlockSpec((7,), lambda i: (0,)),
        ],
        out_specs=pl.BlockSpec((RB, 7), lambda i: (i, 0)),
        out_shape=jax.ShapeDtypeStruct((N, 7), f32),
    )(sacc, hist, wa, ba, wb, bb)


def _pad_conv_edges(edge_index):
    src, dst = edge_index[0], edge_index[1]
    pad = EC_PAD - EC_T
    src2 = src.reshape(NS, EC_T)
    dst2 = dst.reshape(NS, EC_T)
    fill_src = (jnp.arange(pad, dtype=jnp.int32) * 89) % N
    fill_dst = N + (jnp.arange(pad, dtype=jnp.int32) % NS)
    src_pad = jnp.concatenate(
        [src2, jnp.broadcast_to(fill_src, (NS, pad))], axis=1)
    dst_pad = jnp.concatenate(
        [dst2, jnp.broadcast_to(fill_dst, (NS, pad))], axis=1)
    return (src_pad.reshape(NS, EC_CHUNKS, CH),
            dst_pad.reshape(NS, EC_CHUNKS, CH))


def _pad_hist(dst):
    pad = EH_PAD - EH_T
    d = dst.reshape(NC * NS, EH_T)
    fill = N + (jnp.arange(pad, dtype=jnp.int32) % NS)
    d = jnp.concatenate([d, jnp.broadcast_to(fill, (NC * NS, pad))], axis=1)
    return d.reshape(NC, NS, EH_CHUNKS, CH)


def _pad_seg_hist(idx):
    total = NC * NS * SI_PAD
    fill = N + (jnp.arange(total - N, dtype=jnp.int32) % NS)
    d = jnp.concatenate([idx, fill])
    return d.reshape(NC, NS, SI_CHUNKS, CH)


def _pad_seg_scatter(idx):
    fill = N + (jnp.arange(NROW_PAD - N, dtype=jnp.int32) % NS)
    d = jnp.concatenate([idx, fill])
    return d.reshape(NS * SR_CHUNKS, CH)


def kernel(x, edge_index_1, edge_index_2, edge_index_3, index_1, index_2,
           index_3, W1_1, b1_1, W1_2, b1_2, W1_3, b1_3, mlp1_Wa, mlp1_ba,
           mlp1_Wb, mlp1_bb, W2_1, b2_1, W2_2, b2_2, W2_3, b2_3, mlp2_Wa,
           mlp2_ba, mlp2_Wb, mlp2_bb, mlp3_Wa, mlp3_ba, mlp3_Wb, mlp3_bb):
    w1_all = jnp.concatenate([W1_1, W1_2, W1_3], axis=1)
    w2_all = jnp.concatenate([W2_1, W2_2, W2_3], axis=1)
    b1_all = jnp.stack([b1_1, b1_2, b1_3])
    b2_all = jnp.stack([b2_1, b2_2, b2_3])

    edges = [edge_index_1, edge_index_2, edge_index_3]
    srcs, dsts = zip(*[_pad_conv_edges(e) for e in edges])
    src_pad = jnp.stack(srcs)
    dst_pad = jnp.stack(dsts)
    hist_dsts = jnp.stack([_pad_hist(e[1]) for e in edges])
    hist_segs = jnp.stack([_pad_seg_hist(index_1),
                           _pad_seg_hist(index_3)])
    seg_scatter = jnp.stack([_pad_seg_scatter(index_1),
                             _pad_seg_scatter(index_3)])

    hist = _hist_kernel(hist_dsts, hist_segs)
    h1 = _stage1_matmul(x, w1_all)
    g1 = _stage1_scale(h1, hist)
    acc1 = _conv_kernel(g1, src_pad, dst_pad)
    g2 = _stage_epilogue(b1_all, acc1, g1, hist, mlp1_Wa, mlp1_ba,
                         mlp1_Wb, mlp1_bb, w2_all)
    acc2 = _conv_kernel(g2, src_pad, dst_pad)
    h3pad = _stage_epilogue2(b2_all, acc2, g2, hist, mlp2_Wa, mlp2_ba,
                             mlp2_Wb, mlp2_bb)
    sacc = _segmean_kernel(h3pad, seg_scatter)
    return _stage_mlp3(sacc, hist, mlp3_Wa, mlp3_ba, mlp3_Wb, mlp3_bb)
